# baseline (speedup 1.0000x reference)
import functools

import jax
import jax.numpy as jnp
from jax import lax
from jax.experimental import pallas as pl
from jax.experimental.pallas import tpu as pltpu
from jax.experimental.pallas import tpu_sc as plsc

_BATCH = 16384
_HIST = 50
_DIM = 64
_PDIM = 128
_NB = _BATCH * _HIST
_VOCAB = 1000000

_INFO = plsc.get_sparse_core_info()
_NC = _INFO.num_cores
_NS = _INFO.num_subcores
_NW = _NC * _NS

_CHUNK = 128
_TBB = _BATCH // _CHUNK
_NBLK = _NB // _CHUNK
_BLK_PER_W = _NBLK // _NW
_ROWS_PER_W = _NB // _NW


def _lookup_kernel(idx_hbm, tbl_hbm, out_hbm, ibuf, gb0, gb1, ob0, ob1,
                   gs0, gs1, os0, os1):
    wid = lax.axis_index("s") * _NC + lax.axis_index("c")
    wbase = wid * _ROWS_PER_W
    gbase = wid * _BLK_PER_W
    gbs = (gb0, gb1)
    obs = (ob0, ob1)
    gss = (gs0, gs1)
    oss = (os0, os1)
    lanes = lax.iota(jnp.int32, 16)

    pltpu.sync_copy(idx_hbm.at[pl.ds(wbase, _ROWS_PER_W)], ibuf)

    def fire_gather(k, buf, sem):
        pltpu.async_copy(tbl_hbm.at[ibuf.at[pl.ds(k * _CHUNK, _CHUNK)]],
                         buf, sem)

    def absorb_gather(buf, sem):
        pltpu.make_async_copy(tbl_hbm.at[pl.ds(0, _CHUNK)], buf, sem).wait()

    def absorb_wb(buf, sem):
        pltpu.make_async_copy(buf, out_hbm.at[0, :, pl.ds(0, _CHUNK)],
                              sem).wait()

    def transpose(gb, ob):
        def lg_body(lg):
            base_l = lg * 16
            splats = [jnp.full((16,), base_l + j, jnp.int32)
                      for j in range(16)]
            for c0 in range(0, _DIM, 16):
                cvec = lanes + c0
                xs = [gb[base_l + j, pl.ds(c0, 16)] for j in range(16)]
                for j in range(16):
                    plsc.store_scatter(ob, [cvec, splats[j]], xs[j])
        plsc.parallel_loop(0, _CHUNK // 16, unroll=2)(lg_body)

    def writeback(g, ob, sem):
        h = g // _TBB
        tb = g % _TBB
        pltpu.async_copy(ob, out_hbm.at[h, :, pl.ds(tb * _CHUNK, _CHUNK)],
                         sem)

    fire_gather(0, gbs[0], gss[0])
    fire_gather(1, gbs[1], gss[1])

    def body(kk):
        for b in range(2):
            k = kk + b
            absorb_gather(gbs[b], gss[b])

            @pl.when(kk >= 2)
            def _():
                absorb_wb(obs[b], oss[b])

            transpose(gbs[b], obs[b])
            writeback(gbase + k, obs[b], oss[b])

            @pl.when(kk < _BLK_PER_W - 2)
            def _():
                fire_gather(k + 2, gbs[b], gss[b])

    pl.loop(0, _BLK_PER_W, step=2)(body)

    absorb_wb(obs[0], oss[0])
    absorb_wb(obs[1], oss[1])


@jax.jit
def _run(indices, table):
    padded = jnp.pad(table, ((0, 0), (0, _PDIM - _DIM)))
    idx_hm = jnp.transpose(indices).reshape(-1)
    mesh = plsc.VectorSubcoreMesh(core_axis_name="c", subcore_axis_name="s")
    lookup = functools.partial(
        pl.kernel,
        mesh=mesh,
        out_type=jax.ShapeDtypeStruct((_HIST, _DIM, _BATCH), jnp.float32),
        scratch_types=[
            pltpu.VMEM((_ROWS_PER_W,), jnp.int32),
            pltpu.VMEM((_CHUNK, _PDIM), jnp.float32),
            pltpu.VMEM((_CHUNK, _PDIM), jnp.float32),
            pltpu.VMEM((_DIM, _CHUNK), jnp.float32),
            pltpu.VMEM((_DIM, _CHUNK), jnp.float32),
            pltpu.SemaphoreType.DMA,
            pltpu.SemaphoreType.DMA,
            pltpu.SemaphoreType.DMA,
            pltpu.SemaphoreType.DMA,
        ],
        compiler_params=pltpu.CompilerParams(use_tc_tiling_on_sc=True,
                                             needs_layout_passes=False,
                                             disable_bounds_checks=True),
    )(_lookup_kernel)
    out3 = lookup(idx_hm, padded)
    return jnp.transpose(out3, (2, 0, 1))


def kernel(indices, table):
    return _run(indices, table)


# speedup vs baseline: 1.0618x; 1.0618x over previous
import functools

import jax
import jax.numpy as jnp
from jax import lax
from jax.experimental import pallas as pl
from jax.experimental.pallas import tpu as pltpu
from jax.experimental.pallas import tpu_sc as plsc

_BATCH = 16384
_HIST = 50
_DIM = 64
_PDIM = 128
_NB = _BATCH * _HIST
_VOCAB = 1000000

_INFO = plsc.get_sparse_core_info()
_NC = _INFO.num_cores
_NS = _INFO.num_subcores
_NW = _NC * _NS

_CHUNK = 128
_TBB = _BATCH // _CHUNK
_NBLK = _NB // _CHUNK
_BLK_PER_W = _NBLK // _NW
_ROWS_PER_W = _NB // _NW


def _lookup_kernel(idx_hbm, tbl_hbm, out_hbm, ibuf, qbuf, gb0, gb1, ob0, ob1,
                   gs0, gs1, os0, os1):
    wid = lax.axis_index("s") * _NC + lax.axis_index("c")
    wbase = wid * _ROWS_PER_W
    gbase = wid * _BLK_PER_W
    gbs = (gb0, gb1)
    obs = (ob0, ob1)
    gss = (gs0, gs1)
    oss = (os0, os1)
    lanes = lax.iota(jnp.int32, 16)

    pltpu.sync_copy(idx_hbm.at[pl.ds(wbase, _ROWS_PER_W)], ibuf)

    def q_body(i):
        qbuf[pl.ds(i * 16, 16)] = jax.lax.shift_right_logical(
            ibuf[pl.ds(i * 16, 16)], 1)
    plsc.parallel_loop(0, _ROWS_PER_W // 16, unroll=4)(q_body)

    def fire_gather(k, buf, sem):
        pltpu.async_copy(tbl_hbm.at[qbuf.at[pl.ds(k * _CHUNK, _CHUNK)]],
                         buf, sem)

    def absorb_gather(buf, sem):
        pltpu.make_async_copy(tbl_hbm.at[pl.ds(0, _CHUNK)], buf, sem).wait()

    def absorb_wb(buf, sem):
        pltpu.make_async_copy(buf, out_hbm.at[0, :, pl.ds(0, _CHUNK)],
                              sem).wait()

    def transpose(k, gb, ob):
        def lg_body(lg):
            lvec = lanes + lg * 16
            par64 = jax.lax.shift_left(
                ibuf[pl.ds(k * _CHUNK + lg * 16, 16)] & 1, 6)
            for c0 in range(0, _DIM, 16):
                xs = [
                    plsc.load_gather(gb, [lvec, par64 + (c0 + i)])
                    for i in range(16)
                ]
                for i in range(16):
                    ob[c0 + i, pl.ds(lg * 16, 16)] = xs[i]
        plsc.parallel_loop(0, _CHUNK // 16, unroll=1)(lg_body)

    def writeback(g, ob, sem):
        h = g // _TBB
        tb = g % _TBB
        pltpu.async_copy(ob, out_hbm.at[h, :, pl.ds(tb * _CHUNK, _CHUNK)],
                         sem)

    fire_gather(0, gbs[0], gss[0])
    fire_gather(1, gbs[1], gss[1])

    def body(kk):
        for b in range(2):
            k = kk + b
            absorb_gather(gbs[b], gss[b])

            @pl.when(kk >= 2)
            def _():
                absorb_wb(obs[b], oss[b])

            transpose(k, gbs[b], obs[b])
            writeback(gbase + k, obs[b], oss[b])

            @pl.when(kk < _BLK_PER_W - 2)
            def _():
                fire_gather(k + 2, gbs[b], gss[b])

    pl.loop(0, _BLK_PER_W, step=2)(body)

    absorb_wb(obs[0], oss[0])
    absorb_wb(obs[1], oss[1])


@jax.jit
def _run(indices, table):
    paired = table.reshape(_VOCAB // 2, _PDIM)
    idx_hm = jnp.transpose(indices).reshape(-1)
    mesh = plsc.VectorSubcoreMesh(core_axis_name="c", subcore_axis_name="s")
    lookup = functools.partial(
        pl.kernel,
        mesh=mesh,
        out_type=jax.ShapeDtypeStruct((_HIST, _DIM, _BATCH), jnp.float32),
        scratch_types=[
            pltpu.VMEM((_ROWS_PER_W,), jnp.int32),
            pltpu.VMEM((_ROWS_PER_W,), jnp.int32),
            pltpu.VMEM((_CHUNK, _PDIM), jnp.float32),
            pltpu.VMEM((_CHUNK, _PDIM), jnp.float32),
            pltpu.VMEM((_DIM, _CHUNK), jnp.float32),
            pltpu.VMEM((_DIM, _CHUNK), jnp.float32),
            pltpu.SemaphoreType.DMA,
            pltpu.SemaphoreType.DMA,
            pltpu.SemaphoreType.DMA,
            pltpu.SemaphoreType.DMA,
        ],
        compiler_params=pltpu.CompilerParams(use_tc_tiling_on_sc=True,
                                             needs_layout_passes=False,
                                             disable_bounds_checks=True),
    )(_lookup_kernel)
    out3 = lookup(idx_hm, paired)
    return jnp.transpose(out3, (2, 0, 1))


def kernel(indices, table):
    return _run(indices, table)


# speedup vs baseline: 1.1284x; 1.0627x over previous
import functools

import jax
import jax.numpy as jnp
from jax import lax
from jax.experimental import pallas as pl
from jax.experimental.pallas import tpu as pltpu
from jax.experimental.pallas import tpu_sc as plsc

_BATCH = 16384
_HIST = 50
_DIM = 64
_PDIM = 128
_NB = _BATCH * _HIST
_VOCAB = 1000000

_INFO = plsc.get_sparse_core_info()
_NC = _INFO.num_cores
_NS = _INFO.num_subcores
_NW = _NC * _NS

_CHUNK = 128
_TBB = _BATCH // _CHUNK
_NBLK = _NB // _CHUNK
_BLK_PER_W = _NBLK // _NW
_ROWS_PER_W = _NB // _NW


def _lookup_kernel(idx_hbm, tbl_hbm, out_hbm, ibuf, gb0, gb1, ob0, ob1,
                   gs0, gs1, os0, os1):
    wid = lax.axis_index("s") * _NC + lax.axis_index("c")
    wbase = wid * _ROWS_PER_W
    gbase = wid * _BLK_PER_W
    gbs = (gb0, gb1)
    obs = (ob0, ob1)
    gss = (gs0, gs1)
    oss = (os0, os1)
    lanes = lax.iota(jnp.int32, 16)

    pltpu.sync_copy(idx_hbm.at[pl.ds(wbase, _ROWS_PER_W)], ibuf)

    def fire_gather(k, buf, sem):
        pltpu.async_copy(tbl_hbm.at[ibuf.at[pl.ds(k * _CHUNK, _CHUNK)]],
                         buf, sem)

    def absorb_gather(buf, sem):
        pltpu.make_async_copy(tbl_hbm.at[pl.ds(0, _CHUNK)], buf, sem).wait()

    def absorb_wb(buf, sem):
        pltpu.make_async_copy(buf, out_hbm.at[0, :, pl.ds(0, _CHUNK)],
                              sem).wait()

    def transpose(k, gb, ob):
        def lg_body(lg):
            lvec = lanes + lg * 16
            for c0 in range(0, _DIM, 16):
                xs = [
                    plsc.load_gather(
                        gb, [lvec, jnp.full((16,), c0 + i, jnp.int32)])
                    for i in range(16)
                ]
                for i in range(16):
                    ob[c0 + i, pl.ds(lg * 16, 16)] = xs[i]
        plsc.parallel_loop(0, _CHUNK // 16, unroll=1)(lg_body)

    def writeback(g, ob, sem):
        h = g // _TBB
        tb = g % _TBB
        pltpu.async_copy(ob, out_hbm.at[h, :, pl.ds(tb * _CHUNK, _CHUNK)],
                         sem)

    fire_gather(0, gbs[0], gss[0])
    fire_gather(1, gbs[1], gss[1])

    def body(kk):
        for b in range(2):
            k = kk + b
            absorb_gather(gbs[b], gss[b])

            @pl.when(kk >= 2)
            def _():
                absorb_wb(obs[b], oss[b])

            transpose(k, gbs[b], obs[b])
            writeback(gbase + k, obs[b], oss[b])

            @pl.when(kk < _BLK_PER_W - 2)
            def _():
                fire_gather(k + 2, gbs[b], gss[b])

    pl.loop(0, _BLK_PER_W, step=2)(body)

    absorb_wb(obs[0], oss[0])
    absorb_wb(obs[1], oss[1])


@jax.jit
def _run(indices, table):
    padded = jnp.pad(table, ((0, 0), (0, _PDIM - _DIM)))
    idx_hm = jnp.transpose(indices).reshape(-1)
    mesh = plsc.VectorSubcoreMesh(core_axis_name="c", subcore_axis_name="s")
    lookup = functools.partial(
        pl.kernel,
        mesh=mesh,
        out_type=jax.ShapeDtypeStruct((_HIST, _DIM, _BATCH), jnp.float32),
        scratch_types=[
            pltpu.VMEM((_ROWS_PER_W,), jnp.int32),
            pltpu.VMEM((_CHUNK, _PDIM), jnp.float32),
            pltpu.VMEM((_CHUNK, _PDIM), jnp.float32),
            pltpu.VMEM((_DIM, _CHUNK), jnp.float32),
            pltpu.VMEM((_DIM, _CHUNK), jnp.float32),
            pltpu.SemaphoreType.DMA,
            pltpu.SemaphoreType.DMA,
            pltpu.SemaphoreType.DMA,
            pltpu.SemaphoreType.DMA,
        ],
        compiler_params=pltpu.CompilerParams(use_tc_tiling_on_sc=True,
                                             needs_layout_passes=False,
                                             disable_bounds_checks=True),
    )(_lookup_kernel)
    out3 = lookup(idx_hm, padded)
    return jnp.transpose(out3, (2, 0, 1))


def kernel(indices, table):
    return _run(indices, table)
